# baseline (speedup 1.0000x reference)
import functools

import jax
import jax.numpy as jnp
from jax import lax
from jax.experimental import pallas as pl
from jax.experimental.pallas import tpu as pltpu
from jax.experimental.pallas import tpu_sc as plsc

N = 10000
D = 128
E = 320000
P = 128

C = 100
ROWS = E // C
NW = 32
RPT = ROWS // NW
K = 20
NB = RPT // K
NPT = 640
NPAD = 16 * NPT

f32 = jnp.float32


def _sc_scatter_add(m, src3d, dst3d, zrows):
    mesh = plsc.VectorSubcoreMesh(core_axis_name="c", subcore_axis_name="s")

    @functools.partial(
        pl.kernel,
        mesh=mesh,
        out_type=jax.ShapeDtypeStruct((NW, NPT, D), f32),
        scratch_types=[
            pltpu.VMEM((K, C), jnp.int32),
            pltpu.VMEM((K, C), jnp.int32),
            pltpu.VMEM((C, D), f32),
            pltpu.VMEM((C, D), f32),
            pltpu.VMEM_SHARED((NPAD, D), f32),
            pltpu.SemaphoreType.DMA,
            pltpu.SemaphoreType.DMA,
        ],
    )
    def k(m_hbm, src_hbm, dst_hbm, z_hbm, out_hbm,
          src_v, dst_v, rows_a, rows_b, acc, sem_a, sem_b):
        c = lax.axis_index("c")
        s = lax.axis_index("s")
        wid = s * 2 + c
        pltpu.sync_copy(z_hbm, acc.at[pl.ds(s * NPT, NPT)])
        plsc.subcore_barrier()

        def blk_body(blk, carry):
            pltpu.sync_copy(src_hbm.at[wid, blk], src_v)
            pltpu.sync_copy(dst_hbm.at[wid, blk], dst_v)

            pltpu.async_copy(m_hbm.at[src_v.at[0]], rows_a, sem_a)
            pltpu.async_copy(m_hbm.at[src_v.at[1]], rows_b, sem_b)

            def body(j2, carry2):
                j = 2 * j2
                pltpu.make_async_copy(m_hbm.at[src_v.at[j]], rows_a, sem_a).wait()
                pltpu.sync_copy(rows_a, acc.at[dst_v.at[j]], add=True)
                pltpu.async_copy(m_hbm.at[src_v.at[j + 2]], rows_a, sem_a)
                pltpu.make_async_copy(
                    m_hbm.at[src_v.at[j + 1]], rows_b, sem_b).wait()
                pltpu.sync_copy(rows_b, acc.at[dst_v.at[j + 1]], add=True)
                pltpu.async_copy(m_hbm.at[src_v.at[j + 3]], rows_b, sem_b)
                return carry2

            lax.fori_loop(0, K // 2 - 1, body, 0)

            je = K - 2
            pltpu.make_async_copy(m_hbm.at[src_v.at[je]], rows_a, sem_a).wait()
            pltpu.sync_copy(rows_a, acc.at[dst_v.at[je]], add=True)
            pltpu.make_async_copy(m_hbm.at[src_v.at[je + 1]], rows_b, sem_b).wait()
            pltpu.sync_copy(rows_b, acc.at[dst_v.at[je + 1]], add=True)
            return carry

        lax.fori_loop(0, NB, blk_body, 0)
        plsc.subcore_barrier()
        pltpu.sync_copy(acc.at[pl.ds(s * NPT, NPT)], out_hbm.at[c * 16 + s])

    return k(m, src3d, dst3d, zrows)


def _tc_matmul(x, W):
    def body(x_r, w_r, m_r):
        m_r[...] = jnp.dot(x_r[...], w_r[...], preferred_element_type=f32)

    return pl.pallas_call(
        body, out_shape=jax.ShapeDtypeStruct((N, D), f32))(x, W)


def _tc_res(x, Wr, br):
    def body(x_r, wr_r, br_r, res_r):
        res_r[...] = jnp.maximum(
            jnp.dot(x_r[...], wr_r[...], preferred_element_type=f32)
            + br_r[...], 0.0)

    return pl.pallas_call(
        body, out_shape=jax.ShapeDtypeStruct((N, D), f32))(x, Wr, br.reshape(1, D))


def _bn_block(aggs_r, b_r, res_r, g_r, be_r):
    agg = aggs_r[0:N, :] + aggs_r[NPAD:NPAD + N, :]
    h = jnp.maximum(agg + b_r[...], 0.0) + res_r[...]
    mean = jnp.mean(h, axis=0, keepdims=True)
    d = h - mean
    var = jnp.mean(d * d, axis=0, keepdims=True)
    return d * lax.rsqrt(var + 1e-5) * g_r[...] + be_r[...]


def _tc_mid(aggs, b, res, g, be, W1):
    def body(aggs_r, b_r, res_r, g_r, be_r, w_r, m_r, hn_r):
        hn = _bn_block(aggs_r, b_r, res_r, g_r, be_r)
        m_r[...] = jnp.dot(hn, w_r[...], preferred_element_type=f32)
        hn_r[...] = hn

    return pl.pallas_call(
        body,
        out_shape=(jax.ShapeDtypeStruct((N, D), f32),
                   jax.ShapeDtypeStruct((N, D), f32)),
    )(aggs, b, res, g, be, W1)


def _tc_out(aggs, b, res, g, be, Wa, ba, Wt, bt):
    def body(aggs_r, b_r, res_r, g_r, be_r, wa_r, ba_r, wt_r, bt_r, o_r):
        hn = _bn_block(aggs_r, b_r, res_r, g_r, be_r)
        w = jax.nn.sigmoid(
            jnp.dot(hn, wa_r[...], preferred_element_type=f32) + ba_r[...])
        hsum = jnp.sum(w * hn, axis=0, keepdims=True)
        hmax = jnp.max(hn, axis=0, keepdims=True)
        hg = jnp.concatenate([hsum, hmax], axis=1)
        o_r[...] = jnp.dot(hg, wt_r[...], preferred_element_type=f32) + bt_r[...]

    return pl.pallas_call(
        body,
        out_shape=jax.ShapeDtypeStruct((1, P), f32),
    )(aggs, b, res, g, be, Wa, ba, Wt, bt)


def kernel(x, edge_index, W0, b0, Wr0, br0, g0, be0,
           W1, b1, Wr1, br1, g1, be1, Wa, ba, Wt, bt):
    src3d = edge_index[0].reshape(NW, NB, K, C)
    dst3d = edge_index[1].reshape(NW, NB, K, C)
    z = jnp.zeros((NPT, D), f32)

    m0 = _tc_matmul(x, W0)
    aggs0 = _sc_scatter_add(m0, src3d, dst3d, z).reshape(2 * NPAD, D)
    res0 = _tc_res(x, Wr0, br0)
    m1, hn0 = _tc_mid(aggs0, b0.reshape(1, D), res0, g0.reshape(1, D),
                      be0.reshape(1, D), W1)
    aggs1 = _sc_scatter_add(m1, src3d, dst3d, z).reshape(2 * NPAD, D)
    res1 = _tc_res(hn0, Wr1, br1)
    return _tc_out(aggs1, b1.reshape(1, D), res1, g1.reshape(1, D),
                   be1.reshape(1, D), Wa, ba.reshape(1, 1), Wt, bt.reshape(1, P))


# speedup vs baseline: 1.0425x; 1.0425x over previous
import functools

import jax
import jax.numpy as jnp
from jax import lax
from jax.experimental import pallas as pl
from jax.experimental.pallas import tpu as pltpu
from jax.experimental.pallas import tpu_sc as plsc

N = 10000
D = 128
E = 320000
P = 128

C = 125
ROWS = E // C
NW = 32
RPT = ROWS // NW
K = 20
NB = RPT // K
NPT = 640
NPAD = 16 * NPT

f32 = jnp.float32


def _sc_scatter_add(m, src3d, dst3d, zrows):
    mesh = plsc.VectorSubcoreMesh(core_axis_name="c", subcore_axis_name="s")

    @functools.partial(
        pl.kernel,
        mesh=mesh,
        out_type=jax.ShapeDtypeStruct((NW, NPT, D), f32),
        scratch_types=[
            pltpu.VMEM((K, C), jnp.int32),
            pltpu.VMEM((K, C), jnp.int32),
            pltpu.VMEM((C, D), f32),
            pltpu.VMEM((C, D), f32),
            pltpu.VMEM_SHARED((NPAD, D), f32),
            pltpu.SemaphoreType.DMA,
            pltpu.SemaphoreType.DMA,
        ],
    )
    def k(m_hbm, src_hbm, dst_hbm, z_hbm, out_hbm,
          src_v, dst_v, rows_a, rows_b, acc, sem_a, sem_b):
        c = lax.axis_index("c")
        s = lax.axis_index("s")
        wid = s * 2 + c
        pltpu.sync_copy(z_hbm, acc.at[pl.ds(s * NPT, NPT)])
        plsc.subcore_barrier()

        def blk_body(blk, carry):
            pltpu.sync_copy(src_hbm.at[wid, blk], src_v)
            pltpu.sync_copy(dst_hbm.at[wid, blk], dst_v)

            pltpu.async_copy(m_hbm.at[src_v.at[0]], rows_a, sem_a)
            pltpu.async_copy(m_hbm.at[src_v.at[1]], rows_b, sem_b)

            def body(j2, carry2):
                j = 2 * j2
                pltpu.make_async_copy(m_hbm.at[src_v.at[j]], rows_a, sem_a).wait()
                pltpu.sync_copy(rows_a, acc.at[dst_v.at[j]], add=True)
                pltpu.async_copy(m_hbm.at[src_v.at[j + 2]], rows_a, sem_a)
                pltpu.make_async_copy(
                    m_hbm.at[src_v.at[j + 1]], rows_b, sem_b).wait()
                pltpu.sync_copy(rows_b, acc.at[dst_v.at[j + 1]], add=True)
                pltpu.async_copy(m_hbm.at[src_v.at[j + 3]], rows_b, sem_b)
                return carry2

            lax.fori_loop(0, K // 2 - 1, body, 0)

            je = K - 2
            pltpu.make_async_copy(m_hbm.at[src_v.at[je]], rows_a, sem_a).wait()
            pltpu.sync_copy(rows_a, acc.at[dst_v.at[je]], add=True)
            pltpu.make_async_copy(m_hbm.at[src_v.at[je + 1]], rows_b, sem_b).wait()
            pltpu.sync_copy(rows_b, acc.at[dst_v.at[je + 1]], add=True)
            return carry

        lax.fori_loop(0, NB, blk_body, 0)
        plsc.subcore_barrier()
        pltpu.sync_copy(acc.at[pl.ds(s * NPT, NPT)], out_hbm.at[c * 16 + s])

    return k(m, src3d, dst3d, zrows)


def _tc_matmul(x, W):
    def body(x_r, w_r, m_r):
        m_r[...] = jnp.dot(x_r[...], w_r[...], preferred_element_type=f32)

    return pl.pallas_call(
        body, out_shape=jax.ShapeDtypeStruct((N, D), f32))(x, W)


def _tc_res(x, Wr, br):
    def body(x_r, wr_r, br_r, res_r):
        res_r[...] = jnp.maximum(
            jnp.dot(x_r[...], wr_r[...], preferred_element_type=f32)
            + br_r[...], 0.0)

    return pl.pallas_call(
        body, out_shape=jax.ShapeDtypeStruct((N, D), f32))(x, Wr, br.reshape(1, D))


def _bn_block(aggs_r, b_r, res_r, g_r, be_r):
    agg = aggs_r[0:N, :] + aggs_r[NPAD:NPAD + N, :]
    h = jnp.maximum(agg + b_r[...], 0.0) + res_r[...]
    mean = jnp.mean(h, axis=0, keepdims=True)
    d = h - mean
    var = jnp.mean(d * d, axis=0, keepdims=True)
    return d * lax.rsqrt(var + 1e-5) * g_r[...] + be_r[...]


def _tc_mid(aggs, b, res, g, be, W1):
    def body(aggs_r, b_r, res_r, g_r, be_r, w_r, m_r, hn_r):
        hn = _bn_block(aggs_r, b_r, res_r, g_r, be_r)
        m_r[...] = jnp.dot(hn, w_r[...], preferred_element_type=f32)
        hn_r[...] = hn

    return pl.pallas_call(
        body,
        out_shape=(jax.ShapeDtypeStruct((N, D), f32),
                   jax.ShapeDtypeStruct((N, D), f32)),
    )(aggs, b, res, g, be, W1)


def _tc_out(aggs, b, res, g, be, Wa, ba, Wt, bt):
    def body(aggs_r, b_r, res_r, g_r, be_r, wa_r, ba_r, wt_r, bt_r, o_r):
        hn = _bn_block(aggs_r, b_r, res_r, g_r, be_r)
        w = jax.nn.sigmoid(
            jnp.dot(hn, wa_r[...], preferred_element_type=f32) + ba_r[...])
        hsum = jnp.sum(w * hn, axis=0, keepdims=True)
        hmax = jnp.max(hn, axis=0, keepdims=True)
        hg = jnp.concatenate([hsum, hmax], axis=1)
        o_r[...] = jnp.dot(hg, wt_r[...], preferred_element_type=f32) + bt_r[...]

    return pl.pallas_call(
        body,
        out_shape=jax.ShapeDtypeStruct((1, P), f32),
    )(aggs, b, res, g, be, Wa, ba, Wt, bt)


def kernel(x, edge_index, W0, b0, Wr0, br0, g0, be0,
           W1, b1, Wr1, br1, g1, be1, Wa, ba, Wt, bt):
    src3d = edge_index[0].reshape(NW, NB, K, C)
    dst3d = edge_index[1].reshape(NW, NB, K, C)
    z = jnp.zeros((NPT, D), f32)

    m0 = _tc_matmul(x, W0)
    aggs0 = _sc_scatter_add(m0, src3d, dst3d, z).reshape(2 * NPAD, D)
    res0 = _tc_res(x, Wr0, br0)
    m1, hn0 = _tc_mid(aggs0, b0.reshape(1, D), res0, g0.reshape(1, D),
                      be0.reshape(1, D), W1)
    aggs1 = _sc_scatter_add(m1, src3d, dst3d, z).reshape(2 * NPAD, D)
    res1 = _tc_res(hn0, Wr1, br1)
    return _tc_out(aggs1, b1.reshape(1, D), res1, g1.reshape(1, D),
                   be1.reshape(1, D), Wa, ba.reshape(1, 1), Wt, bt.reshape(1, P))


# speedup vs baseline: 1.0865x; 1.0423x over previous
import functools

import jax
import jax.numpy as jnp
from jax import lax
from jax.experimental import pallas as pl
from jax.experimental.pallas import tpu as pltpu
from jax.experimental.pallas import tpu_sc as plsc

N = 10000
D = 128
E = 320000
P = 128

C = 125
ROWS = E // C
NW = 32
RPT = ROWS // NW
K = 40
NB = RPT // K
NPT = 640
NPAD = 16 * NPT

f32 = jnp.float32


def _sc_scatter_add(m, src3d, dst3d, zrows):
    mesh = plsc.VectorSubcoreMesh(core_axis_name="c", subcore_axis_name="s")

    @functools.partial(
        pl.kernel,
        mesh=mesh,
        out_type=jax.ShapeDtypeStruct((NW, NPT, D), f32),
        scratch_types=[
            pltpu.VMEM((K, C), jnp.int32),
            pltpu.VMEM((K, C), jnp.int32),
            pltpu.VMEM((C, D), f32),
            pltpu.VMEM((C, D), f32),
            pltpu.VMEM_SHARED((NPAD, D), f32),
            pltpu.SemaphoreType.DMA,
            pltpu.SemaphoreType.DMA,
        ],
    )
    def k(m_hbm, src_hbm, dst_hbm, z_hbm, out_hbm,
          src_v, dst_v, rows_a, rows_b, acc, sem_a, sem_b):
        c = lax.axis_index("c")
        s = lax.axis_index("s")
        wid = s * 2 + c
        pltpu.sync_copy(z_hbm, acc.at[pl.ds(s * NPT, NPT)])
        plsc.subcore_barrier()

        def blk_body(blk, carry):
            pltpu.sync_copy(src_hbm.at[wid, blk], src_v)
            pltpu.sync_copy(dst_hbm.at[wid, blk], dst_v)

            pltpu.async_copy(m_hbm.at[src_v.at[0]], rows_a, sem_a)
            pltpu.async_copy(m_hbm.at[src_v.at[1]], rows_b, sem_b)

            def body(j2, carry2):
                j = 2 * j2
                pltpu.make_async_copy(m_hbm.at[src_v.at[j]], rows_a, sem_a).wait()
                pltpu.sync_copy(rows_a, acc.at[dst_v.at[j]], add=True)
                pltpu.async_copy(m_hbm.at[src_v.at[j + 2]], rows_a, sem_a)
                pltpu.make_async_copy(
                    m_hbm.at[src_v.at[j + 1]], rows_b, sem_b).wait()
                pltpu.sync_copy(rows_b, acc.at[dst_v.at[j + 1]], add=True)
                pltpu.async_copy(m_hbm.at[src_v.at[j + 3]], rows_b, sem_b)
                return carry2

            lax.fori_loop(0, K // 2 - 1, body, 0)

            je = K - 2
            pltpu.make_async_copy(m_hbm.at[src_v.at[je]], rows_a, sem_a).wait()
            pltpu.sync_copy(rows_a, acc.at[dst_v.at[je]], add=True)
            pltpu.make_async_copy(m_hbm.at[src_v.at[je + 1]], rows_b, sem_b).wait()
            pltpu.sync_copy(rows_b, acc.at[dst_v.at[je + 1]], add=True)
            return carry

        lax.fori_loop(0, NB, blk_body, 0)
        plsc.subcore_barrier()
        pltpu.sync_copy(acc.at[pl.ds(s * NPT, NPT)], out_hbm.at[c * 16 + s])

    return k(m, src3d, dst3d, zrows)


def _tc_matmul(x, W):
    def body(x_r, w_r, m_r):
        m_r[...] = jnp.dot(x_r[...], w_r[...], preferred_element_type=f32)

    return pl.pallas_call(
        body, out_shape=jax.ShapeDtypeStruct((N, D), f32))(x, W)


def _tc_res(x, Wr, br):
    def body(x_r, wr_r, br_r, res_r):
        res_r[...] = jnp.maximum(
            jnp.dot(x_r[...], wr_r[...], preferred_element_type=f32)
            + br_r[...], 0.0)

    return pl.pallas_call(
        body, out_shape=jax.ShapeDtypeStruct((N, D), f32))(x, Wr, br.reshape(1, D))


def _bn_block(aggs_r, b_r, res_r, g_r, be_r):
    agg = aggs_r[0:N, :] + aggs_r[NPAD:NPAD + N, :]
    h = jnp.maximum(agg + b_r[...], 0.0) + res_r[...]
    mean = jnp.mean(h, axis=0, keepdims=True)
    d = h - mean
    var = jnp.mean(d * d, axis=0, keepdims=True)
    return d * lax.rsqrt(var + 1e-5) * g_r[...] + be_r[...]


def _tc_mid(aggs, b, res, g, be, W1):
    def body(aggs_r, b_r, res_r, g_r, be_r, w_r, m_r, hn_r):
        hn = _bn_block(aggs_r, b_r, res_r, g_r, be_r)
        m_r[...] = jnp.dot(hn, w_r[...], preferred_element_type=f32)
        hn_r[...] = hn

    return pl.pallas_call(
        body,
        out_shape=(jax.ShapeDtypeStruct((N, D), f32),
                   jax.ShapeDtypeStruct((N, D), f32)),
    )(aggs, b, res, g, be, W1)


def _tc_out(aggs, b, res, g, be, Wa, ba, Wt, bt):
    def body(aggs_r, b_r, res_r, g_r, be_r, wa_r, ba_r, wt_r, bt_r, o_r):
        hn = _bn_block(aggs_r, b_r, res_r, g_r, be_r)
        w = jax.nn.sigmoid(
            jnp.dot(hn, wa_r[...], preferred_element_type=f32) + ba_r[...])
        hsum = jnp.sum(w * hn, axis=0, keepdims=True)
        hmax = jnp.max(hn, axis=0, keepdims=True)
        hg = jnp.concatenate([hsum, hmax], axis=1)
        o_r[...] = jnp.dot(hg, wt_r[...], preferred_element_type=f32) + bt_r[...]

    return pl.pallas_call(
        body,
        out_shape=jax.ShapeDtypeStruct((1, P), f32),
    )(aggs, b, res, g, be, Wa, ba, Wt, bt)


def kernel(x, edge_index, W0, b0, Wr0, br0, g0, be0,
           W1, b1, Wr1, br1, g1, be1, Wa, ba, Wt, bt):
    src3d = edge_index[0].reshape(NW, NB, K, C)
    dst3d = edge_index[1].reshape(NW, NB, K, C)
    z = jnp.zeros((NPT, D), f32)

    m0 = _tc_matmul(x, W0)
    aggs0 = _sc_scatter_add(m0, src3d, dst3d, z).reshape(2 * NPAD, D)
    res0 = _tc_res(x, Wr0, br0)
    m1, hn0 = _tc_mid(aggs0, b0.reshape(1, D), res0, g0.reshape(1, D),
                      be0.reshape(1, D), W1)
    aggs1 = _sc_scatter_add(m1, src3d, dst3d, z).reshape(2 * NPAD, D)
    res1 = _tc_res(hn0, Wr1, br1)
    return _tc_out(aggs1, b1.reshape(1, D), res1, g1.reshape(1, D),
                   be1.reshape(1, D), Wa, ba.reshape(1, 1), Wt, bt.reshape(1, P))
